# baseline (speedup 1.0000x reference)
import functools

import jax
import jax.numpy as jnp
from jax import lax
from jax.experimental import pallas as pl
from jax.experimental.pallas import tpu as pltpu
from jax.experimental.pallas import tpu_sc as plsc

FVS = 512
MASK_ID = FVS * FVS
D = 1024
L = 16
NBUF = 2
HI_MASK = 0xFFFF0000


def _pack_table(t):
    tb = t.astype(jnp.bfloat16)
    u = lax.bitcast_convert_type(tb, jnp.uint16).astype(jnp.uint32)
    u3 = u.reshape(-1, D // 32, 2, L)
    p = (u3[:, :, 0, :] | (u3[:, :, 1, :] << 16)).reshape(-1, D // 2)
    return lax.bitcast_convert_type(p, jnp.int32)


def _sc_lookup(ids, tab, n_workers, chunk):
    n = ids.shape[0]
    per_w = n // n_workers
    n_chunks = per_w // chunk
    n_pairs = n_chunks // NBUF
    mesh = plsc.VectorSubcoreMesh(
        core_axis_name="c", subcore_axis_name="s", num_cores=2, num_subcores=16)

    @functools.partial(
        pl.kernel,
        out_type=jax.ShapeDtypeStruct((n, D), jnp.float32),
        mesh=mesh,
        scratch_types=[
            pltpu.VMEM((per_w,), jnp.int32),
            [pltpu.VMEM((2 * chunk,), jnp.int32) for _ in range(NBUF)],
            [pltpu.VMEM((2 * chunk, D // 2), jnp.int32) for _ in range(NBUF)],
            [pltpu.VMEM((chunk, D), jnp.float32) for _ in range(NBUF)],
            [pltpu.SemaphoreType.DMA for _ in range(NBUF)],
            [pltpu.SemaphoreType.DMA for _ in range(NBUF)],
        ],
    )
    def k(ids_hbm, tab_hbm, out_hbm,
          ids_v, idxs, rows, outb, gsem, ssem):
        n_cores = lax.axis_size("c")
        wid = lax.axis_index("s") * n_cores + lax.axis_index("c")
        base = wid * per_w
        pltpu.sync_copy(ids_hbm.at[pl.ds(base, per_w)], ids_v)

        def compute_idx(c, b):
            tok0 = c * chunk
            for j in range(chunk // L):
                v = ids_v[pl.ds(tok0 + j * L, L)]
                m = v == MASK_ID
                idxs[b][pl.ds(j * L, L)] = jnp.where(m, FVS, v & (FVS - 1))
                idxs[b][pl.ds(chunk + j * L, L)] = jnp.where(
                    m, 2 * FVS + 1, ((v >> 9) & (FVS - 1)) + (FVS + 1))

        def start_gather(b):
            pltpu.async_copy(tab_hbm.at[idxs[b]], rows[b], gsem[b])

        def wait_gather(b):
            pltpu.make_async_copy(tab_hbm.at[idxs[b]], rows[b], gsem[b]).wait()

        def add(b):
            hi_mask = jnp.int32(-65536)

            @plsc.parallel_loop(0, chunk * (D // 32), unroll=8)
            def _(i):
                t = i >> 5
                kk = i & 31
                s = pl.ds(kk * L, L)
                w0 = rows[b][t, s]
                w1 = rows[b][t + chunk, s]
                lo = (lax.bitcast_convert_type(w0 << 16, jnp.float32)
                      + lax.bitcast_convert_type(w1 << 16, jnp.float32))
                hi = (lax.bitcast_convert_type(w0 & hi_mask, jnp.float32)
                      + lax.bitcast_convert_type(w1 & hi_mask, jnp.float32))
                outb[b][t, pl.ds(kk * 32, L)] = lo
                outb[b][t, pl.ds(kk * 32 + L, L)] = hi

        def start_scatter(c, b):
            pltpu.async_copy(
                outb[b], out_hbm.at[pl.ds(base + c * chunk, chunk)], ssem[b])

        def wait_scatter(b):
            pltpu.make_async_copy(
                outb[b], out_hbm.at[pl.ds(base, chunk)], ssem[b]).wait()

        for b in range(NBUF):
            compute_idx(b, b)
            start_gather(b)

        def pair_body(i, carry):
            for b in range(NBUF):
                c = NBUF * i + b
                wait_gather(b)

                @pl.when(i >= 1)
                def _():
                    wait_scatter(b)

                add(b)
                start_scatter(c, b)

                @pl.when(c + NBUF < n_chunks)
                def _():
                    compute_idx(c + NBUF, b)
                    start_gather(b)
            return carry

        lax.fori_loop(0, n_pairs, pair_body, 0, unroll=False)
        for b in range(NBUF):
            wait_scatter(b)

    return k(ids, tab)


def kernel(input_ids, e0, e1, mask_token_embed):
    orig_shape = input_ids.shape
    ids = input_ids.reshape(-1).astype(jnp.int32)
    t0 = _pack_table(jnp.concatenate([e0, mask_token_embed], axis=0))
    t1 = _pack_table(jnp.concatenate([e1, jnp.zeros_like(mask_token_embed)], axis=0))
    tab = jnp.concatenate([t0, t1], axis=0)
    out = _sc_lookup(ids, tab, n_workers=32, chunk=16)
    return out.reshape(orig_shape + (D,))


# speedup vs baseline: 1.0187x; 1.0187x over previous
import functools

import jax
import jax.numpy as jnp
from jax import lax
from jax.experimental import pallas as pl
from jax.experimental.pallas import tpu as pltpu
from jax.experimental.pallas import tpu_sc as plsc

FVS = 512
MASK_ID = FVS * FVS
D = 1024
L = 16
NBUF = 2
NGBUF = 4
HI_MASK = 0xFFFF0000


def _pack_table(t):
    tb = t.astype(jnp.bfloat16)
    u = lax.bitcast_convert_type(tb, jnp.uint16).astype(jnp.uint32)
    u3 = u.reshape(-1, D // 32, 2, L)
    p = (u3[:, :, 0, :] | (u3[:, :, 1, :] << 16)).reshape(-1, D // 2)
    return lax.bitcast_convert_type(p, jnp.int32)


def _sc_lookup(ids, tab, n_workers, chunk):
    n = ids.shape[0]
    per_w = n // n_workers
    n_chunks = per_w // chunk
    n_pairs = n_chunks // NBUF
    mesh = plsc.VectorSubcoreMesh(
        core_axis_name="c", subcore_axis_name="s", num_cores=2, num_subcores=16)

    @functools.partial(
        pl.kernel,
        out_type=jax.ShapeDtypeStruct((n, D), jnp.float32),
        mesh=mesh,
        scratch_types=[
            pltpu.VMEM((per_w,), jnp.int32),
            [pltpu.VMEM((2 * chunk,), jnp.int32) for _ in range(NGBUF)],
            [pltpu.VMEM((2 * chunk, D // 2), jnp.int32) for _ in range(NGBUF)],
            [pltpu.VMEM((chunk, D), jnp.float32) for _ in range(NBUF)],
            [pltpu.SemaphoreType.DMA for _ in range(NGBUF)],
            [pltpu.SemaphoreType.DMA for _ in range(NBUF)],
        ],
    )
    def k(ids_hbm, tab_hbm, out_hbm,
          ids_v, idxs, rows, outb, gsem, ssem):
        n_cores = lax.axis_size("c")
        wid = lax.axis_index("s") * n_cores + lax.axis_index("c")
        base = wid * per_w
        pltpu.sync_copy(ids_hbm.at[pl.ds(base, per_w)], ids_v)

        def compute_idx(c, b):
            tok0 = c * chunk
            for j in range(chunk // L):
                v = ids_v[pl.ds(tok0 + j * L, L)]
                m = v == MASK_ID
                idxs[b][pl.ds(j * L, L)] = jnp.where(m, FVS, v & (FVS - 1))
                idxs[b][pl.ds(chunk + j * L, L)] = jnp.where(
                    m, 2 * FVS + 1, ((v >> 9) & (FVS - 1)) + (FVS + 1))

        def start_gather(b):
            pltpu.async_copy(tab_hbm.at[idxs[b]], rows[b], gsem[b])

        def wait_gather(b):
            pltpu.make_async_copy(tab_hbm.at[idxs[b]], rows[b], gsem[b]).wait()

        def add(rb, ob):
            hi_mask = jnp.int32(-65536)

            @plsc.parallel_loop(0, chunk * (D // 32), unroll=8)
            def _(i):
                t = i >> 5
                kk = i & 31
                s = pl.ds(kk * L, L)
                w0 = rows[rb][t, s]
                w1 = rows[rb][t + chunk, s]
                lo = (lax.bitcast_convert_type(w0 << 16, jnp.float32)
                      + lax.bitcast_convert_type(w1 << 16, jnp.float32))
                hi = (lax.bitcast_convert_type(w0 & hi_mask, jnp.float32)
                      + lax.bitcast_convert_type(w1 & hi_mask, jnp.float32))
                outb[ob][t, pl.ds(kk * 32, L)] = lo
                outb[ob][t, pl.ds(kk * 32 + L, L)] = hi

        def start_scatter(c, b):
            pltpu.async_copy(
                outb[b], out_hbm.at[pl.ds(base + c * chunk, chunk)], ssem[b])

        def wait_scatter(b):
            pltpu.make_async_copy(
                outb[b], out_hbm.at[pl.ds(base, chunk)], ssem[b]).wait()

        for b in range(NGBUF - 1):
            compute_idx(b, b)
            start_gather(b)

        def quad_body(i, carry):
            for j in range(NGBUF):
                c = NGBUF * i + j
                ob = j % NBUF
                wait_gather(j)

                @pl.when(c + NGBUF - 1 < n_chunks)
                def _():
                    compute_idx(c + NGBUF - 1, (j + NGBUF - 1) % NGBUF)
                    start_gather((j + NGBUF - 1) % NGBUF)

                @pl.when(c >= NBUF)
                def _():
                    wait_scatter(ob)

                add(j, ob)
                start_scatter(c, ob)
            return carry

        lax.fori_loop(0, n_chunks // NGBUF, quad_body, 0, unroll=False)
        for b in range(NBUF):
            wait_scatter(b)

    return k(ids, tab)


def kernel(input_ids, e0, e1, mask_token_embed):
    orig_shape = input_ids.shape
    ids = input_ids.reshape(-1).astype(jnp.int32)
    t0 = _pack_table(jnp.concatenate([e0, mask_token_embed], axis=0))
    t1 = _pack_table(jnp.concatenate([e1, jnp.zeros_like(mask_token_embed)], axis=0))
    tab = jnp.concatenate([t0, t1], axis=0)
    out = _sc_lookup(ids, tab, n_workers=32, chunk=16)
    return out.reshape(orig_shape + (D,))
